# baseline (speedup 1.0000x reference)
import functools

import jax
import jax.numpy as jnp
from jax import lax
from jax.experimental import pallas as pl
from jax.experimental.pallas import tpu as pltpu
from jax.experimental.pallas import tpu_sc as plsc

N = 10000
E = 320000
DF = 128
DH = 64
DT = 2
EPS = 1e-5

R = 400
NB = N // R

NC = 2
NS = 16
NW = NC * NS
CS = 128
CH = 80
EPT = CH * CS
E_PAD = NW * EPT
TRASH = N
N_TRASH = 10240
RPT = N_TRASH // NS
FPT = N // NS


def _tc1_body(x_ref, w1t_ref, b1_ref, g1_ref, be1_ref, feat_ref, h_scr, acc):
    p = pl.program_id(0)
    i = pl.program_id(1)

    @pl.when((p == 0) & (i == 0))
    def _init():
        acc[...] = jnp.zeros_like(acc)

    @pl.when(p == 0)
    def _accumulate():
        h = jnp.dot(x_ref[...], w1t_ref[...],
                    preferred_element_type=jnp.float32) + b1_ref[...]
        h_scr[pl.ds(i * R, R), :] = h
        acc[0:1, :] += jnp.sum(h, axis=0, keepdims=True)
        acc[1:2, :] += jnp.sum(h * h, axis=0, keepdims=True)

    @pl.when((p == 1) & (i == 0))
    def _stats():
        mean = acc[0:1, :] * (1.0 / N)
        var = acc[1:2, :] * (1.0 / N) - mean * mean
        acc[2:3, :] = mean
        acc[3:4, :] = lax.rsqrt(var + EPS) * g1_ref[...]

    @pl.when(p == 1)
    def _normalize():
        h = h_scr[pl.ds(i * R, R), :]
        f = (h - acc[2:3, :]) * acc[3:4, :] + be1_ref[...]
        feat_ref[...] = jnp.maximum(f, 0.0)


def _tc1(x, w1t, b1, g1, be1):
    return pl.pallas_call(
        _tc1_body,
        grid=(2, NB),
        in_specs=[
            pl.BlockSpec((R, DF), lambda p, i: ((1 - p) * i, 0)),
            pl.BlockSpec((DF, DH), lambda p, i: (0, 0)),
            pl.BlockSpec((1, DH), lambda p, i: (0, 0)),
            pl.BlockSpec((1, DH), lambda p, i: (0, 0)),
            pl.BlockSpec((1, DH), lambda p, i: (0, 0)),
        ],
        out_specs=pl.BlockSpec((R, DH), lambda p, i: (p * i, 0)),
        out_shape=jax.ShapeDtypeStruct((N, DH), jnp.float32),
        scratch_shapes=[
            pltpu.VMEM((N, DH), jnp.float32),
            pltpu.VMEM((8, DH), jnp.float32),
        ],
    )(x, w1t, b1, g1, be1)


def _sc_body(src_hbm, dst_hbm, feat_hbm, zeros_hbm, out_hbm,
             src_v, dst_v, rows_a, rows_b, agg_sh, feat_sh,
             sem_a, sem_b, ssem_a, ssem_b):
    c = lax.axis_index("c")
    s = lax.axis_index("s")
    wid = s * NC + c

    pltpu.sync_copy(zeros_hbm, agg_sh.at[pl.ds(s * RPT, RPT)])
    pltpu.sync_copy(feat_hbm.at[pl.ds(s * FPT, FPT)],
                    feat_sh.at[pl.ds(s * FPT, FPT)])
    pltpu.sync_copy(src_hbm.at[wid], src_v)
    pltpu.sync_copy(dst_hbm.at[wid], dst_v)
    plsc.subcore_barrier()

    pltpu.async_copy(feat_sh.at[src_v.at[0]], rows_a, sem_a)
    pltpu.async_copy(feat_sh.at[src_v.at[1]], rows_b, sem_b)

    def body(g, carry):
        j0 = g * 2
        pltpu.make_async_copy(feat_sh.at[src_v.at[j0]], rows_a, sem_a).wait()
        pltpu.async_copy(rows_a, agg_sh.at[dst_v.at[j0]], ssem_a, add=True)
        pltpu.make_async_copy(feat_sh.at[src_v.at[j0 + 1]], rows_b,
                              sem_b).wait()
        pltpu.async_copy(rows_b, agg_sh.at[dst_v.at[j0 + 1]], ssem_b,
                         add=True)

        @pl.when(j0 + 2 < CH)
        def _():
            pltpu.make_async_copy(rows_a, agg_sh.at[dst_v.at[j0]],
                                  ssem_a).wait()
            pltpu.async_copy(feat_sh.at[src_v.at[j0 + 2]], rows_a, sem_a)
            pltpu.make_async_copy(rows_b, agg_sh.at[dst_v.at[j0 + 1]],
                                  ssem_b).wait()
            pltpu.async_copy(feat_sh.at[src_v.at[j0 + 3]], rows_b, sem_b)

        return carry

    lax.fori_loop(0, CH // 2, body, 0)
    pltpu.make_async_copy(rows_a, agg_sh.at[dst_v.at[CH - 2]], ssem_a).wait()
    pltpu.make_async_copy(rows_b, agg_sh.at[dst_v.at[CH - 1]], ssem_b).wait()
    plsc.subcore_barrier()
    pltpu.sync_copy(agg_sh.at[pl.ds(s * RPT, RPT)],
                    out_hbm.at[c, pl.ds(s * RPT, RPT)])


@functools.cache
def _build_sc_scatter():
    return functools.partial(
        pl.kernel,
        out_type=jax.ShapeDtypeStruct((NC, N_TRASH, DH), jnp.float32),
        mesh=plsc.VectorSubcoreMesh(core_axis_name="c", subcore_axis_name="s",
                                    num_cores=NC, num_subcores=NS),
        scratch_types=[
            pltpu.VMEM((CH, CS), jnp.int32),
            pltpu.VMEM((CH, CS), jnp.int32),
            pltpu.VMEM((CS, DH), jnp.float32),
            pltpu.VMEM((CS, DH), jnp.float32),
            pltpu.VMEM_SHARED((N_TRASH, DH), jnp.float32),
            pltpu.VMEM_SHARED((N, DH), jnp.float32),
            pltpu.SemaphoreType.DMA,
            pltpu.SemaphoreType.DMA,
            pltpu.SemaphoreType.DMA,
            pltpu.SemaphoreType.DMA,
        ],
        compiler_params=pltpu.CompilerParams(use_tc_tiling_on_sc=False),
    )(_sc_body)


def _tc2_body(feat_ref, agg_ref, wct_ref, bc_ref, gc_ref, bec_ref,
              l0t_ref, l0b_ref, l1t_ref, l1b_ref, out_ref, h_scr, acc):
    p = pl.program_id(0)
    i = pl.program_id(1)

    @pl.when((p == 0) & (i == 0))
    def _init():
        acc[...] = jnp.zeros_like(acc)

    @pl.when(p == 0)
    def _accumulate():
        u = feat_ref[...] + agg_ref[0] + agg_ref[1]
        h = jnp.dot(u, wct_ref[...],
                    preferred_element_type=jnp.float32) + bc_ref[...]
        h_scr[pl.ds(i * R, R), :] = h
        acc[0:1, :] += jnp.sum(h, axis=0, keepdims=True)
        acc[1:2, :] += jnp.sum(h * h, axis=0, keepdims=True)

    @pl.when((p == 1) & (i == 0))
    def _stats():
        mean = acc[0:1, :] * (1.0 / N)
        var = acc[1:2, :] * (1.0 / N) - mean * mean
        acc[2:3, :] = mean
        acc[3:4, :] = lax.rsqrt(var + EPS) * gc_ref[...]

    @pl.when(p == 1)
    def _finish():
        h = h_scr[pl.ds(i * R, R), :]
        f2 = jnp.maximum((h - acc[2:3, :]) * acc[3:4, :] + bec_ref[...], 0.0)
        np0 = jnp.dot(feat_ref[...], l0t_ref[...],
                      preferred_element_type=jnp.float32) + l0b_ref[...]
        np1 = jnp.dot(f2, l1t_ref[...],
                      preferred_element_type=jnp.float32) + l1b_ref[...]
        logits = np0 + np1
        m = jnp.max(logits, axis=1, keepdims=True)
        z = logits - m
        out_ref[...] = z - jnp.log(jnp.sum(jnp.exp(z), axis=1, keepdims=True))


def _tc2(feature, agg2, wct, bc, gc, bec, l0t, l0b, l1t, l1b):
    return pl.pallas_call(
        _tc2_body,
        grid=(2, NB),
        in_specs=[
            pl.BlockSpec((R, DH), lambda p, i: (i, 0)),
            pl.BlockSpec((NC, R, DH), lambda p, i: (0, (1 - p) * i, 0)),
            pl.BlockSpec((DH, DH), lambda p, i: (0, 0)),
            pl.BlockSpec((1, DH), lambda p, i: (0, 0)),
            pl.BlockSpec((1, DH), lambda p, i: (0, 0)),
            pl.BlockSpec((1, DH), lambda p, i: (0, 0)),
            pl.BlockSpec((DH, DT), lambda p, i: (0, 0)),
            pl.BlockSpec((1, DT), lambda p, i: (0, 0)),
            pl.BlockSpec((DH, DT), lambda p, i: (0, 0)),
            pl.BlockSpec((1, DT), lambda p, i: (0, 0)),
        ],
        out_specs=pl.BlockSpec((R, DT), lambda p, i: (p * i, 0)),
        out_shape=jax.ShapeDtypeStruct((N, DT), jnp.float32),
        scratch_shapes=[
            pltpu.VMEM((N, DH), jnp.float32),
            pltpu.VMEM((8, DH), jnp.float32),
        ],
    )(feature, agg2, wct, bc, gc, bec, l0t, l0b, l1t, l1b)


def kernel(x, edge_index, batch, W1, b1, g1, be1, Wc, bc, gc, bec,
           lin0_W, lin0_b, lin1_W, lin1_b):
    feature = _tc1(x, W1.T,
                   b1.reshape(1, DH), g1.reshape(1, DH), be1.reshape(1, DH))

    src = edge_index[0].astype(jnp.int32)
    dst = edge_index[1].astype(jnp.int32)
    pad = E_PAD - E
    src_p = jnp.concatenate(
        [src, jnp.zeros((pad,), jnp.int32)]).reshape(NW, CH, CS)
    dst_p = jnp.concatenate(
        [dst, jnp.full((pad,), TRASH, jnp.int32)]).reshape(NW, CH, CS)
    zeros = jnp.zeros((RPT, DH), jnp.float32)
    agg2 = _build_sc_scatter()(src_p, dst_p, feature, zeros)

    return _tc2(feature, agg2, Wc.T,
                bc.reshape(1, DH), gc.reshape(1, DH), bec.reshape(1, DH),
                lin0_W.T, lin0_b.reshape(1, DT),
                lin1_W.T, lin1_b.reshape(1, DT))


# speedup vs baseline: 1.1995x; 1.1995x over previous
import functools

import jax
import jax.numpy as jnp
from jax import lax
from jax.experimental import pallas as pl
from jax.experimental.pallas import tpu as pltpu
from jax.experimental.pallas import tpu_sc as plsc

N = 10000
E = 320000
DF = 128
DH = 64
DT = 2
EPS = 1e-5

R = 1000
NB = N // R

NC = 2
NS = 16
NW = NC * NS
CS = 128
CH = 80
EPT = CH * CS
E_PAD = NW * EPT
TRASH = N
N_TRASH = 10240
RPT = N_TRASH // NS
FPT = N // NS


def _tc1_body(x_ref, w1t_ref, b1_ref, g1_ref, be1_ref, feat_ref, h_scr, acc):
    p = pl.program_id(0)
    i = pl.program_id(1)

    @pl.when((p == 0) & (i == 0))
    def _init():
        acc[...] = jnp.zeros_like(acc)

    @pl.when(p == 0)
    def _accumulate():
        h = jnp.dot(x_ref[...], w1t_ref[...],
                    preferred_element_type=jnp.float32) + b1_ref[...]
        h_scr[pl.ds(i * R, R), :] = h
        acc[0:1, :] += jnp.sum(h, axis=0, keepdims=True)
        acc[1:2, :] += jnp.sum(h * h, axis=0, keepdims=True)

    @pl.when((p == 1) & (i == 0))
    def _stats():
        mean = acc[0:1, :] * (1.0 / N)
        var = acc[1:2, :] * (1.0 / N) - mean * mean
        acc[2:3, :] = mean
        acc[3:4, :] = lax.rsqrt(var + EPS) * g1_ref[...]

    @pl.when(p == 1)
    def _normalize():
        h = h_scr[pl.ds(i * R, R), :]
        f = (h - acc[2:3, :]) * acc[3:4, :] + be1_ref[...]
        feat_ref[...] = jnp.maximum(f, 0.0)


def _tc1(x, w1t, b1, g1, be1):
    return pl.pallas_call(
        _tc1_body,
        grid=(2, NB),
        in_specs=[
            pl.BlockSpec((R, DF), lambda p, i: ((1 - p) * i, 0)),
            pl.BlockSpec((DF, DH), lambda p, i: (0, 0)),
            pl.BlockSpec((1, DH), lambda p, i: (0, 0)),
            pl.BlockSpec((1, DH), lambda p, i: (0, 0)),
            pl.BlockSpec((1, DH), lambda p, i: (0, 0)),
        ],
        out_specs=pl.BlockSpec((R, DH), lambda p, i: (p * i, 0)),
        out_shape=jax.ShapeDtypeStruct((N, DH), jnp.float32),
        scratch_shapes=[
            pltpu.VMEM((N, DH), jnp.float32),
            pltpu.VMEM((8, DH), jnp.float32),
        ],
    )(x, w1t, b1, g1, be1)


def _sc_body(src_hbm, dst_hbm, feat_hbm, zeros_hbm, out_hbm,
             src_v, dst_v, rows_a, rows_b, agg_sh, feat_sh, sem_a, sem_b):
    c = lax.axis_index("c")
    s = lax.axis_index("s")
    wid = s * NC + c

    pltpu.sync_copy(zeros_hbm, agg_sh.at[pl.ds(s * RPT, RPT)])
    pltpu.sync_copy(feat_hbm.at[pl.ds(s * FPT, FPT)],
                    feat_sh.at[pl.ds(s * FPT, FPT)])
    pltpu.sync_copy(src_hbm.at[wid], src_v)
    pltpu.sync_copy(dst_hbm.at[wid], dst_v)
    plsc.subcore_barrier()

    pltpu.async_copy(feat_sh.at[src_v.at[0]], rows_a, sem_a)

    def body(g, carry):
        j0 = g * 2
        pltpu.make_async_copy(feat_sh.at[src_v.at[j0]], rows_a, sem_a).wait()
        pltpu.async_copy(feat_sh.at[src_v.at[j0 + 1]], rows_b, sem_b)
        pltpu.sync_copy(rows_a, agg_sh.at[dst_v.at[j0]], add=True)
        pltpu.make_async_copy(feat_sh.at[src_v.at[j0 + 1]], rows_b,
                              sem_b).wait()

        @pl.when(j0 + 2 < CH)
        def _():
            pltpu.async_copy(feat_sh.at[src_v.at[j0 + 2]], rows_a, sem_a)

        pltpu.sync_copy(rows_b, agg_sh.at[dst_v.at[j0 + 1]], add=True)
        return carry

    lax.fori_loop(0, CH // 2, body, 0)
    plsc.subcore_barrier()
    pltpu.sync_copy(agg_sh.at[pl.ds(s * RPT, RPT)],
                    out_hbm.at[c, pl.ds(s * RPT, RPT)])


@functools.cache
def _build_sc_scatter():
    return functools.partial(
        pl.kernel,
        out_type=jax.ShapeDtypeStruct((NC, N_TRASH, DH), jnp.float32),
        mesh=plsc.VectorSubcoreMesh(core_axis_name="c", subcore_axis_name="s",
                                    num_cores=NC, num_subcores=NS),
        scratch_types=[
            pltpu.VMEM((CH, CS), jnp.int32),
            pltpu.VMEM((CH, CS), jnp.int32),
            pltpu.VMEM((CS, DH), jnp.float32),
            pltpu.VMEM((CS, DH), jnp.float32),
            pltpu.VMEM_SHARED((N_TRASH, DH), jnp.float32),
            pltpu.VMEM_SHARED((N, DH), jnp.float32),
            pltpu.SemaphoreType.DMA,
            pltpu.SemaphoreType.DMA,
        ],
        compiler_params=pltpu.CompilerParams(use_tc_tiling_on_sc=False),
    )(_sc_body)


def _tc2_body(feat_ref, agg_ref, wct_ref, bc_ref, gc_ref, bec_ref,
              l0t_ref, l0b_ref, l1t_ref, l1b_ref, out_ref, h_scr, acc):
    p = pl.program_id(0)
    i = pl.program_id(1)

    @pl.when((p == 0) & (i == 0))
    def _init():
        acc[...] = jnp.zeros_like(acc)

    @pl.when(p == 0)
    def _accumulate():
        u = feat_ref[...] + agg_ref[0] + agg_ref[1]
        h = jnp.dot(u, wct_ref[...],
                    preferred_element_type=jnp.float32) + bc_ref[...]
        h_scr[pl.ds(i * R, R), :] = h
        acc[0:1, :] += jnp.sum(h, axis=0, keepdims=True)
        acc[1:2, :] += jnp.sum(h * h, axis=0, keepdims=True)

    @pl.when((p == 1) & (i == 0))
    def _stats():
        mean = acc[0:1, :] * (1.0 / N)
        var = acc[1:2, :] * (1.0 / N) - mean * mean
        acc[2:3, :] = mean
        acc[3:4, :] = lax.rsqrt(var + EPS) * gc_ref[...]

    @pl.when(p == 1)
    def _finish():
        h = h_scr[pl.ds(i * R, R), :]
        f2 = jnp.maximum((h - acc[2:3, :]) * acc[3:4, :] + bec_ref[...], 0.0)
        np0 = jnp.dot(feat_ref[...], l0t_ref[...],
                      preferred_element_type=jnp.float32) + l0b_ref[...]
        np1 = jnp.dot(f2, l1t_ref[...],
                      preferred_element_type=jnp.float32) + l1b_ref[...]
        logits = np0 + np1
        m = jnp.max(logits, axis=1, keepdims=True)
        z = logits - m
        out_ref[...] = z - jnp.log(jnp.sum(jnp.exp(z), axis=1, keepdims=True))


def _tc2(feature, agg2, wct, bc, gc, bec, l0t, l0b, l1t, l1b):
    return pl.pallas_call(
        _tc2_body,
        grid=(2, NB),
        in_specs=[
            pl.BlockSpec((R, DH), lambda p, i: (i, 0)),
            pl.BlockSpec((NC, R, DH), lambda p, i: (0, (1 - p) * i, 0)),
            pl.BlockSpec((DH, DH), lambda p, i: (0, 0)),
            pl.BlockSpec((1, DH), lambda p, i: (0, 0)),
            pl.BlockSpec((1, DH), lambda p, i: (0, 0)),
            pl.BlockSpec((1, DH), lambda p, i: (0, 0)),
            pl.BlockSpec((DH, DT), lambda p, i: (0, 0)),
            pl.BlockSpec((1, DT), lambda p, i: (0, 0)),
            pl.BlockSpec((DH, DT), lambda p, i: (0, 0)),
            pl.BlockSpec((1, DT), lambda p, i: (0, 0)),
        ],
        out_specs=pl.BlockSpec((R, DT), lambda p, i: (p * i, 0)),
        out_shape=jax.ShapeDtypeStruct((N, DT), jnp.float32),
        scratch_shapes=[
            pltpu.VMEM((N, DH), jnp.float32),
            pltpu.VMEM((8, DH), jnp.float32),
        ],
    )(feature, agg2, wct, bc, gc, bec, l0t, l0b, l1t, l1b)


def kernel(x, edge_index, batch, W1, b1, g1, be1, Wc, bc, gc, bec,
           lin0_W, lin0_b, lin1_W, lin1_b):
    feature = _tc1(x, W1.T,
                   b1.reshape(1, DH), g1.reshape(1, DH), be1.reshape(1, DH))

    src = edge_index[0].astype(jnp.int32)
    dst = edge_index[1].astype(jnp.int32)
    pad = E_PAD - E
    src_p = jnp.concatenate(
        [src, jnp.zeros((pad,), jnp.int32)]).reshape(NW, CH, CS)
    dst_p = jnp.concatenate(
        [dst, jnp.full((pad,), TRASH, jnp.int32)]).reshape(NW, CH, CS)
    zeros = jnp.zeros((RPT, DH), jnp.float32)
    agg2 = _build_sc_scatter()(src_p, dst_p, feature, zeros)

    return _tc2(feature, agg2, Wc.T,
                bc.reshape(1, DH), gc.reshape(1, DH), bec.reshape(1, DH),
                lin0_W.T, lin0_b.reshape(1, DT),
                lin1_W.T, lin1_b.reshape(1, DT))


# speedup vs baseline: 1.2776x; 1.0651x over previous
import functools

import jax
import jax.numpy as jnp
from jax import lax
from jax.experimental import pallas as pl
from jax.experimental.pallas import tpu as pltpu
from jax.experimental.pallas import tpu_sc as plsc

N = 10000
E = 320000
DF = 128
DH = 64
DT = 2
EPS = 1e-5

R = 2000
NB = N // R

NC = 2
NS = 16
NW = NC * NS
CS = 128
CH = 80
EPT = CH * CS
E_PAD = NW * EPT
TRASH = N
N_TRASH = 10240
RPT = N_TRASH // NS
FPT = N // NS


def _tc1_body(x_ref, w1t_ref, b1_ref, g1_ref, be1_ref, feat_ref, h_scr, acc):
    p = pl.program_id(0)
    i = pl.program_id(1)

    @pl.when((p == 0) & (i == 0))
    def _init():
        acc[...] = jnp.zeros_like(acc)

    @pl.when(p == 0)
    def _accumulate():
        h = jnp.dot(x_ref[...], w1t_ref[...],
                    preferred_element_type=jnp.float32) + b1_ref[...]
        h_scr[pl.ds(i * R, R), :] = h
        acc[0:1, :] += jnp.sum(h, axis=0, keepdims=True)
        acc[1:2, :] += jnp.sum(h * h, axis=0, keepdims=True)

    @pl.when((p == 1) & (i == 0))
    def _stats():
        mean = acc[0:1, :] * (1.0 / N)
        var = acc[1:2, :] * (1.0 / N) - mean * mean
        acc[2:3, :] = mean
        acc[3:4, :] = lax.rsqrt(var + EPS) * g1_ref[...]

    @pl.when(p == 1)
    def _normalize():
        h = h_scr[pl.ds(i * R, R), :]
        f = (h - acc[2:3, :]) * acc[3:4, :] + be1_ref[...]
        feat_ref[...] = jnp.maximum(f, 0.0)


def _tc1(x, w1t, b1, g1, be1):
    return pl.pallas_call(
        _tc1_body,
        grid=(2, NB),
        in_specs=[
            pl.BlockSpec((R, DF), lambda p, i: ((1 - p) * i, 0)),
            pl.BlockSpec((DF, DH), lambda p, i: (0, 0)),
            pl.BlockSpec((1, DH), lambda p, i: (0, 0)),
            pl.BlockSpec((1, DH), lambda p, i: (0, 0)),
            pl.BlockSpec((1, DH), lambda p, i: (0, 0)),
        ],
        out_specs=pl.BlockSpec((R, DH), lambda p, i: (p * i, 0)),
        out_shape=jax.ShapeDtypeStruct((N, DH), jnp.float32),
        scratch_shapes=[
            pltpu.VMEM((N, DH), jnp.float32),
            pltpu.VMEM((8, DH), jnp.float32),
        ],
    )(x, w1t, b1, g1, be1)


def _sc_body(src_hbm, dst_hbm, feat_hbm, zeros_hbm, out_hbm,
             src_v, dst_v, rows_a, rows_b, agg_sh, feat_sh, sem_a, sem_b):
    c = lax.axis_index("c")
    s = lax.axis_index("s")
    wid = s * NC + c

    pltpu.sync_copy(zeros_hbm, agg_sh.at[pl.ds(s * RPT, RPT)])
    pltpu.sync_copy(feat_hbm.at[pl.ds(s * FPT, FPT)],
                    feat_sh.at[pl.ds(s * FPT, FPT)])
    pltpu.sync_copy(src_hbm.at[wid], src_v)
    pltpu.sync_copy(dst_hbm.at[wid], dst_v)
    plsc.subcore_barrier()

    pltpu.async_copy(feat_sh.at[src_v.at[0]], rows_a, sem_a)

    def body(g, carry):
        j0 = g * 2
        pltpu.make_async_copy(feat_sh.at[src_v.at[j0]], rows_a, sem_a).wait()
        pltpu.async_copy(feat_sh.at[src_v.at[j0 + 1]], rows_b, sem_b)
        pltpu.sync_copy(rows_a, agg_sh.at[dst_v.at[j0]], add=True)
        pltpu.make_async_copy(feat_sh.at[src_v.at[j0 + 1]], rows_b,
                              sem_b).wait()

        @pl.when(j0 + 2 < CH)
        def _():
            pltpu.async_copy(feat_sh.at[src_v.at[j0 + 2]], rows_a, sem_a)

        pltpu.sync_copy(rows_b, agg_sh.at[dst_v.at[j0 + 1]], add=True)
        return carry

    lax.fori_loop(0, CH // 2, body, 0)
    plsc.subcore_barrier()
    pltpu.sync_copy(agg_sh.at[pl.ds(s * RPT, RPT)],
                    out_hbm.at[c, pl.ds(s * RPT, RPT)])


@functools.cache
def _build_sc_scatter():
    return functools.partial(
        pl.kernel,
        out_type=jax.ShapeDtypeStruct((NC, N_TRASH, DH), jnp.float32),
        mesh=plsc.VectorSubcoreMesh(core_axis_name="c", subcore_axis_name="s",
                                    num_cores=NC, num_subcores=NS),
        scratch_types=[
            pltpu.VMEM((CH, CS), jnp.int32),
            pltpu.VMEM((CH, CS), jnp.int32),
            pltpu.VMEM((CS, DH), jnp.float32),
            pltpu.VMEM((CS, DH), jnp.float32),
            pltpu.VMEM_SHARED((N_TRASH, DH), jnp.float32),
            pltpu.VMEM_SHARED((N, DH), jnp.float32),
            pltpu.SemaphoreType.DMA,
            pltpu.SemaphoreType.DMA,
        ],
        compiler_params=pltpu.CompilerParams(use_tc_tiling_on_sc=False),
    )(_sc_body)


def _tc2_body(feat_ref, agg_ref, wct_ref, bc_ref, gc_ref, bec_ref,
              l0t_ref, l0b_ref, l1t_ref, l1b_ref, out_ref, h_scr, acc):
    p = pl.program_id(0)
    i = pl.program_id(1)

    @pl.when((p == 0) & (i == 0))
    def _init():
        acc[...] = jnp.zeros_like(acc)

    @pl.when(p == 0)
    def _accumulate():
        u = feat_ref[...] + agg_ref[0] + agg_ref[1]
        h = jnp.dot(u, wct_ref[...],
                    preferred_element_type=jnp.float32) + bc_ref[...]
        h_scr[pl.ds(i * R, R), :] = h
        acc[0:1, :] += jnp.sum(h, axis=0, keepdims=True)
        acc[1:2, :] += jnp.sum(h * h, axis=0, keepdims=True)

    @pl.when((p == 1) & (i == 0))
    def _stats():
        mean = acc[0:1, :] * (1.0 / N)
        var = acc[1:2, :] * (1.0 / N) - mean * mean
        acc[2:3, :] = mean
        acc[3:4, :] = lax.rsqrt(var + EPS) * gc_ref[...]

    @pl.when(p == 1)
    def _finish():
        h = h_scr[pl.ds(i * R, R), :]
        f2 = jnp.maximum((h - acc[2:3, :]) * acc[3:4, :] + bec_ref[...], 0.0)
        np0 = jnp.dot(feat_ref[...], l0t_ref[...],
                      preferred_element_type=jnp.float32) + l0b_ref[...]
        np1 = jnp.dot(f2, l1t_ref[...],
                      preferred_element_type=jnp.float32) + l1b_ref[...]
        logits = np0 + np1
        m = jnp.max(logits, axis=1, keepdims=True)
        z = logits - m
        out_ref[...] = z - jnp.log(jnp.sum(jnp.exp(z), axis=1, keepdims=True))


def _tc2(feature, agg2, wct, bc, gc, bec, l0t, l0b, l1t, l1b):
    return pl.pallas_call(
        _tc2_body,
        grid=(2, NB),
        in_specs=[
            pl.BlockSpec((R, DH), lambda p, i: (i, 0)),
            pl.BlockSpec((NC, R, DH), lambda p, i: (0, (1 - p) * i, 0)),
            pl.BlockSpec((DH, DH), lambda p, i: (0, 0)),
            pl.BlockSpec((1, DH), lambda p, i: (0, 0)),
            pl.BlockSpec((1, DH), lambda p, i: (0, 0)),
            pl.BlockSpec((1, DH), lambda p, i: (0, 0)),
            pl.BlockSpec((DH, DT), lambda p, i: (0, 0)),
            pl.BlockSpec((1, DT), lambda p, i: (0, 0)),
            pl.BlockSpec((DH, DT), lambda p, i: (0, 0)),
            pl.BlockSpec((1, DT), lambda p, i: (0, 0)),
        ],
        out_specs=pl.BlockSpec((R, DT), lambda p, i: (p * i, 0)),
        out_shape=jax.ShapeDtypeStruct((N, DT), jnp.float32),
        scratch_shapes=[
            pltpu.VMEM((N, DH), jnp.float32),
            pltpu.VMEM((8, DH), jnp.float32),
        ],
    )(feature, agg2, wct, bc, gc, bec, l0t, l0b, l1t, l1b)


def kernel(x, edge_index, batch, W1, b1, g1, be1, Wc, bc, gc, bec,
           lin0_W, lin0_b, lin1_W, lin1_b):
    feature = _tc1(x, W1.T,
                   b1.reshape(1, DH), g1.reshape(1, DH), be1.reshape(1, DH))

    src = edge_index[0].astype(jnp.int32)
    dst = edge_index[1].astype(jnp.int32)
    pad = E_PAD - E
    src_p = jnp.concatenate(
        [src, jnp.zeros((pad,), jnp.int32)]).reshape(NW, CH, CS)
    dst_p = jnp.concatenate(
        [dst, jnp.full((pad,), TRASH, jnp.int32)]).reshape(NW, CH, CS)
    zeros = jnp.zeros((RPT, DH), jnp.float32)
    agg2 = _build_sc_scatter()(src_p, dst_p, feature, zeros)

    return _tc2(feature, agg2, Wc.T,
                bc.reshape(1, DH), gc.reshape(1, DH), bec.reshape(1, DH),
                lin0_W.T, lin0_b.reshape(1, DT),
                lin1_W.T, lin1_b.reshape(1, DT))


# speedup vs baseline: 1.3080x; 1.0238x over previous
import functools

import jax
import jax.numpy as jnp
from jax import lax
from jax.experimental import pallas as pl
from jax.experimental.pallas import tpu as pltpu
from jax.experimental.pallas import tpu_sc as plsc

N = 10000
E = 320000
DF = 128
DH = 64
DT = 2
EPS = 1e-5

R = 5000
NB = N // R

NC = 2
NS = 16
NW = NC * NS
CS = 128
CH = 80
EPT = CH * CS
E_PAD = NW * EPT
TRASH = N
N_TRASH = 10240
RPT = N_TRASH // NS
FPT = N // NS


def _tc1_body(x_ref, w1t_ref, b1_ref, g1_ref, be1_ref, feat_ref, h_scr, acc):
    p = pl.program_id(0)
    i = pl.program_id(1)

    @pl.when((p == 0) & (i == 0))
    def _init():
        acc[...] = jnp.zeros_like(acc)

    @pl.when(p == 0)
    def _accumulate():
        h = jnp.dot(x_ref[...], w1t_ref[...],
                    preferred_element_type=jnp.float32) + b1_ref[...]
        h_scr[pl.ds(i * R, R), :] = h
        acc[0:1, :] += jnp.sum(h, axis=0, keepdims=True)
        acc[1:2, :] += jnp.sum(h * h, axis=0, keepdims=True)

    @pl.when((p == 1) & (i == 0))
    def _stats():
        mean = acc[0:1, :] * (1.0 / N)
        var = acc[1:2, :] * (1.0 / N) - mean * mean
        acc[2:3, :] = mean
        acc[3:4, :] = lax.rsqrt(var + EPS) * g1_ref[...]

    @pl.when(p == 1)
    def _normalize():
        h = h_scr[pl.ds(i * R, R), :]
        f = (h - acc[2:3, :]) * acc[3:4, :] + be1_ref[...]
        feat_ref[...] = jnp.maximum(f, 0.0)


def _tc1(x, w1t, b1, g1, be1):
    return pl.pallas_call(
        _tc1_body,
        grid=(2, NB),
        in_specs=[
            pl.BlockSpec((R, DF), lambda p, i: ((1 - p) * i, 0)),
            pl.BlockSpec((DF, DH), lambda p, i: (0, 0)),
            pl.BlockSpec((1, DH), lambda p, i: (0, 0)),
            pl.BlockSpec((1, DH), lambda p, i: (0, 0)),
            pl.BlockSpec((1, DH), lambda p, i: (0, 0)),
        ],
        out_specs=pl.BlockSpec((R, DH), lambda p, i: (p * i, 0)),
        out_shape=jax.ShapeDtypeStruct((N, DH), jnp.float32),
        scratch_shapes=[
            pltpu.VMEM((N, DH), jnp.float32),
            pltpu.VMEM((8, DH), jnp.float32),
        ],
    )(x, w1t, b1, g1, be1)


def _sc_body(src_hbm, dst_hbm, feat_hbm, zeros_hbm, out_hbm,
             src_v, dst_v, rows_a, rows_b, agg_sh, feat_sh, sem_a, sem_b):
    c = lax.axis_index("c")
    s = lax.axis_index("s")
    wid = s * NC + c

    pltpu.sync_copy(zeros_hbm, agg_sh.at[pl.ds(s * RPT, RPT)])
    pltpu.sync_copy(feat_hbm.at[pl.ds(s * FPT, FPT)],
                    feat_sh.at[pl.ds(s * FPT, FPT)])
    pltpu.sync_copy(src_hbm.at[wid], src_v)
    pltpu.sync_copy(dst_hbm.at[wid], dst_v)
    plsc.subcore_barrier()

    pltpu.async_copy(feat_sh.at[src_v.at[0]], rows_a, sem_a)

    def body(g, carry):
        j0 = g * 2
        pltpu.make_async_copy(feat_sh.at[src_v.at[j0]], rows_a, sem_a).wait()
        pltpu.async_copy(feat_sh.at[src_v.at[j0 + 1]], rows_b, sem_b)
        pltpu.sync_copy(rows_a, agg_sh.at[dst_v.at[j0]], add=True)
        pltpu.make_async_copy(feat_sh.at[src_v.at[j0 + 1]], rows_b,
                              sem_b).wait()

        @pl.when(j0 + 2 < CH)
        def _():
            pltpu.async_copy(feat_sh.at[src_v.at[j0 + 2]], rows_a, sem_a)

        pltpu.sync_copy(rows_b, agg_sh.at[dst_v.at[j0 + 1]], add=True)
        return carry

    lax.fori_loop(0, CH // 2, body, 0)
    plsc.subcore_barrier()
    pltpu.sync_copy(agg_sh.at[pl.ds(s * RPT, RPT)],
                    out_hbm.at[c, pl.ds(s * RPT, RPT)])


@functools.cache
def _build_sc_scatter():
    return functools.partial(
        pl.kernel,
        out_type=jax.ShapeDtypeStruct((NC, N_TRASH, DH), jnp.float32),
        mesh=plsc.VectorSubcoreMesh(core_axis_name="c", subcore_axis_name="s",
                                    num_cores=NC, num_subcores=NS),
        scratch_types=[
            pltpu.VMEM((CH, CS), jnp.int32),
            pltpu.VMEM((CH, CS), jnp.int32),
            pltpu.VMEM((CS, DH), jnp.float32),
            pltpu.VMEM((CS, DH), jnp.float32),
            pltpu.VMEM_SHARED((N_TRASH, DH), jnp.float32),
            pltpu.VMEM_SHARED((N, DH), jnp.float32),
            pltpu.SemaphoreType.DMA,
            pltpu.SemaphoreType.DMA,
        ],
        compiler_params=pltpu.CompilerParams(use_tc_tiling_on_sc=False),
    )(_sc_body)


def _tc2_body(feat_ref, agg_ref, wct_ref, bc_ref, gc_ref, bec_ref,
              l0t_ref, l0b_ref, l1t_ref, l1b_ref, out_ref, h_scr, acc):
    p = pl.program_id(0)
    i = pl.program_id(1)

    @pl.when((p == 0) & (i == 0))
    def _init():
        acc[...] = jnp.zeros_like(acc)

    @pl.when(p == 0)
    def _accumulate():
        u = feat_ref[...] + agg_ref[0] + agg_ref[1]
        h = jnp.dot(u, wct_ref[...],
                    preferred_element_type=jnp.float32) + bc_ref[...]
        h_scr[pl.ds(i * R, R), :] = h
        acc[0:1, :] += jnp.sum(h, axis=0, keepdims=True)
        acc[1:2, :] += jnp.sum(h * h, axis=0, keepdims=True)

    @pl.when((p == 1) & (i == 0))
    def _stats():
        mean = acc[0:1, :] * (1.0 / N)
        var = acc[1:2, :] * (1.0 / N) - mean * mean
        acc[2:3, :] = mean
        acc[3:4, :] = lax.rsqrt(var + EPS) * gc_ref[...]

    @pl.when(p == 1)
    def _finish():
        h = h_scr[pl.ds(i * R, R), :]
        f2 = jnp.maximum((h - acc[2:3, :]) * acc[3:4, :] + bec_ref[...], 0.0)
        np0 = jnp.dot(feat_ref[...], l0t_ref[...],
                      preferred_element_type=jnp.float32) + l0b_ref[...]
        np1 = jnp.dot(f2, l1t_ref[...],
                      preferred_element_type=jnp.float32) + l1b_ref[...]
        logits = np0 + np1
        m = jnp.max(logits, axis=1, keepdims=True)
        z = logits - m
        out_ref[...] = z - jnp.log(jnp.sum(jnp.exp(z), axis=1, keepdims=True))


def _tc2(feature, agg2, wct, bc, gc, bec, l0t, l0b, l1t, l1b):
    return pl.pallas_call(
        _tc2_body,
        grid=(2, NB),
        in_specs=[
            pl.BlockSpec((R, DH), lambda p, i: (i, 0)),
            pl.BlockSpec((NC, R, DH), lambda p, i: (0, (1 - p) * i, 0)),
            pl.BlockSpec((DH, DH), lambda p, i: (0, 0)),
            pl.BlockSpec((1, DH), lambda p, i: (0, 0)),
            pl.BlockSpec((1, DH), lambda p, i: (0, 0)),
            pl.BlockSpec((1, DH), lambda p, i: (0, 0)),
            pl.BlockSpec((DH, DT), lambda p, i: (0, 0)),
            pl.BlockSpec((1, DT), lambda p, i: (0, 0)),
            pl.BlockSpec((DH, DT), lambda p, i: (0, 0)),
            pl.BlockSpec((1, DT), lambda p, i: (0, 0)),
        ],
        out_specs=pl.BlockSpec((R, DT), lambda p, i: (p * i, 0)),
        out_shape=jax.ShapeDtypeStruct((N, DT), jnp.float32),
        scratch_shapes=[
            pltpu.VMEM((N, DH), jnp.float32),
            pltpu.VMEM((8, DH), jnp.float32),
        ],
    )(feature, agg2, wct, bc, gc, bec, l0t, l0b, l1t, l1b)


def kernel(x, edge_index, batch, W1, b1, g1, be1, Wc, bc, gc, bec,
           lin0_W, lin0_b, lin1_W, lin1_b):
    feature = _tc1(x, W1.T,
                   b1.reshape(1, DH), g1.reshape(1, DH), be1.reshape(1, DH))

    src = edge_index[0].astype(jnp.int32)
    dst = edge_index[1].astype(jnp.int32)
    pad = E_PAD - E
    src_p = jnp.concatenate(
        [src, jnp.zeros((pad,), jnp.int32)]).reshape(NW, CH, CS)
    dst_p = jnp.concatenate(
        [dst, jnp.full((pad,), TRASH, jnp.int32)]).reshape(NW, CH, CS)
    zeros = jnp.zeros((RPT, DH), jnp.float32)
    agg2 = _build_sc_scatter()(src_p, dst_p, feature, zeros)

    return _tc2(feature, agg2, Wc.T,
                bc.reshape(1, DH), gc.reshape(1, DH), bec.reshape(1, DH),
                lin0_W.T, lin0_b.reshape(1, DT),
                lin1_W.T, lin1_b.reshape(1, DT))
